# baseline (speedup 1.0000x reference)
import functools

import jax
import jax.numpy as jnp
from jax import lax
from jax.experimental import pallas as pl
from jax.experimental.pallas import tpu as pltpu
from jax.experimental.pallas import tpu_sc as plsc

NUM_SETS = 1000000
EMBED_DIM = 64
HIDDEN_DIM = 128
BATCH = 16384
TEAM = 6

NC, NS = 2, 16
NW = NC * NS
TEAMS_PER_W = BATCH // NW
TEAMS_PER_CHUNK = 16
CHUNKS = TEAMS_PER_W // TEAMS_PER_CHUNK
IDX_PER_CHUNK = TEAMS_PER_CHUNK * TEAM
LANES = 16
PHYS_ROWS = NUM_SETS // 2


def _sc_pool(pidx3, h643, table2):
    mesh = plsc.VectorSubcoreMesh(core_axis_name="c", subcore_axis_name="s")

    @functools.partial(
        pl.kernel,
        out_type=jax.ShapeDtypeStruct((EMBED_DIM, BATCH), jnp.float32),
        mesh=mesh,
        scratch_types=[
            pltpu.VMEM((CHUNKS, IDX_PER_CHUNK), jnp.int32),
            pltpu.VMEM((CHUNKS, IDX_PER_CHUNK), jnp.int32),
            pltpu.VMEM((IDX_PER_CHUNK, 2 * EMBED_DIM), jnp.float32),
            pltpu.VMEM((EMBED_DIM, TEAMS_PER_W), jnp.float32),
            pltpu.SemaphoreType.DMA,
        ],
        compiler_params=pltpu.CompilerParams(
            use_tc_tiling_on_sc=False, needs_layout_passes=False
        ),
    )
    def k(pidx_hbm, h64_hbm, table_hbm, out_hbm, pidx_v, h_v, rows_v, out_v, sem):
        wid = lax.axis_index("s") * NC + lax.axis_index("c")
        pltpu.sync_copy(pidx_hbm.at[wid], pidx_v)
        pltpu.sync_copy(h64_hbm.at[wid], h_v)

        iota = lax.iota(jnp.int32, LANES)
        rowvecs = [iota * TEAM + r for r in range(TEAM)]

        def chunk_body(j, carry):
            pltpu.async_copy(table_hbm.at[pidx_v.at[j]], rows_v, sem).wait()
            jv = jnp.full((LANES,), 0, jnp.int32) + j
            h64 = [plsc.load_gather(h_v, [jv, rowvecs[r]]) for r in range(TEAM)]
            for c in range(EMBED_DIM):
                acc = plsc.load_gather(rows_v, [rowvecs[0], h64[0] + c])
                for r in range(1, TEAM):
                    acc = acc + plsc.load_gather(rows_v, [rowvecs[r], h64[r] + c])
                out_v[c, pl.ds(j * TEAMS_PER_CHUNK, LANES)] = acc * (1.0 / TEAM)
            return carry

        lax.fori_loop(0, CHUNKS, chunk_body, 0)
        pltpu.sync_copy(out_v, out_hbm.at[:, pl.ds(wid * TEAMS_PER_W, TEAMS_PER_W)])

    return k(pidx3, h643, table2)


def _tc_mlp_t(xt, w1, b1, w2, b2):
    bb = 2048

    def body(x_ref, w1_ref, b1_ref, w2_ref, b2_ref, o_ref):
        h = jnp.dot(w1_ref[...], x_ref[...], preferred_element_type=jnp.float32)
        h = jnp.maximum(h + b1_ref[...], 0.0)
        o_ref[...] = (
            jnp.dot(w2_ref[...], h, preferred_element_type=jnp.float32) + b2_ref[...]
        )

    return pl.pallas_call(
        body,
        grid=(BATCH // bb,),
        in_specs=[
            pl.BlockSpec((EMBED_DIM, bb), lambda i: (0, i)),
            pl.BlockSpec((HIDDEN_DIM, EMBED_DIM), lambda i: (0, 0)),
            pl.BlockSpec((HIDDEN_DIM, 1), lambda i: (0, 0)),
            pl.BlockSpec((1, HIDDEN_DIM), lambda i: (0, 0)),
            pl.BlockSpec((1, 1), lambda i: (0, 0)),
        ],
        out_specs=pl.BlockSpec((1, bb), lambda i: (0, i)),
        out_shape=jax.ShapeDtypeStruct((1, BATCH), jnp.float32),
    )(xt, w1, b1, w2, b2)


def kernel(team_indices, embedding, fc1_w, fc1_b, fc2_w, fc2_b):
    idx = team_indices.astype(jnp.int32)
    pidx3 = (idx >> 1).reshape(NW, CHUNKS, IDX_PER_CHUNK)
    h643 = ((idx & 1) * EMBED_DIM).reshape(NW, CHUNKS, IDX_PER_CHUNK)
    table2 = embedding.reshape(PHYS_ROWS, 2 * EMBED_DIM)
    pooled_t = _sc_pool(pidx3, h643, table2)
    out = _tc_mlp_t(
        pooled_t,
        fc1_w,
        fc1_b.reshape(HIDDEN_DIM, 1),
        fc2_w,
        fc2_b.reshape(1, 1),
    )
    return out[0]
